# baseline (speedup 1.0000x reference)
import functools

import jax
import jax.numpy as jnp
from jax import lax
from jax.experimental import pallas as pl
from jax.experimental.pallas import tpu as pltpu

_CONTRACT_LAST = (((1,), (1,)), ((), ()))
_NUM_CORES = 2
_CHUNK_B = 16
_DEPTH = 3


def _excite_scale(x_chunk_ref, w1_ref, w2_ref, o_chunk_ref, inv_hw):
    cb = x_chunk_ref.shape[0]
    means = jnp.sum(x_chunk_ref[...], axis=-1, dtype=jnp.float32) * inv_hw
    hid = lax.dot_general(
        means, w1_ref[...], _CONTRACT_LAST, preferred_element_type=jnp.float32
    )
    hid = jnp.maximum(hid, 0.0)
    gate = lax.dot_general(
        w2_ref[...], hid, _CONTRACT_LAST, preferred_element_type=jnp.float32
    )
    gate = jnp.clip(gate + 3.0, 0.0, 6.0) * (1.0 / 6.0)
    gate = gate.astype(o_chunk_ref.dtype)
    for b in range(cb):
        o_chunk_ref[b] = x_chunk_ref[b] * gate[:, b : b + 1]


def _se_manual(
    x_hbm, w1_ref, w2_ref, o_hbm, in_buf, out_buf, in_sems, out_sems,
    *, inv_hw, sizes, per_core, depth,
):
    core = pl.program_id(0)
    base = core * per_core
    offs = []
    off = 0
    for s in sizes:
        offs.append(off)
        off += s
    n_chunks = len(sizes)

    def in_copy(k, slot):
        return pltpu.make_async_copy(
            x_hbm.at[pl.ds(base + offs[k], sizes[k])],
            in_buf.at[slot, pl.ds(0, sizes[k])],
            in_sems.at[slot],
        )

    def out_copy(k, slot):
        return pltpu.make_async_copy(
            out_buf.at[slot, pl.ds(0, sizes[k])],
            o_hbm.at[pl.ds(base + offs[k], sizes[k])],
            out_sems.at[slot],
        )

    for k in range(min(depth, n_chunks)):
        in_copy(k, k % depth).start()

    for k in range(n_chunks):
        s = k % depth
        in_copy(k, s).wait()
        if k >= depth:
            out_copy(k - depth, s).wait()
        _excite_scale(
            in_buf.at[s, pl.ds(0, sizes[k])],
            w1_ref,
            w2_ref,
            out_buf.at[s, pl.ds(0, sizes[k])],
            inv_hw,
        )
        out_copy(k, s).start()
        if k + depth < n_chunks:
            in_copy(k + depth, s).start()

    for k in range(max(n_chunks - depth, 0), n_chunks):
        out_copy(k, k % depth).wait()


def _chunk_schedule(per_core):
    edge = _CHUNK_B // 2
    mid = per_core - 4 * edge
    if mid >= 0 and mid % _CHUNK_B == 0:
        return [edge, edge] + [_CHUNK_B] * (mid // _CHUNK_B) + [edge, edge]
    if per_core > 0 and per_core % _CHUNK_B == 0:
        return [_CHUNK_B] * (per_core // _CHUNK_B)
    return None


def _se_auto(x_ref, w1_ref, w2_ref, o_ref, *, inv_hw):
    _excite_scale(x_ref, w1_ref, w2_ref, o_ref, inv_hw)


def kernel(x_nchw, w1, w2):
    B, C, H, W = x_nchw.shape
    HW = H * W
    x = x_nchw.reshape(B, C, HW)
    inv_hw = 1.0 / HW

    per_core = B // _NUM_CORES
    sizes = _chunk_schedule(per_core)
    if B % _NUM_CORES == 0 and sizes is not None:
        depth = min(_DEPTH, len(sizes))
        out = pl.pallas_call(
            functools.partial(
                _se_manual,
                inv_hw=inv_hw,
                sizes=sizes,
                per_core=per_core,
                depth=depth,
            ),
            out_shape=jax.ShapeDtypeStruct((B, C, HW), x.dtype),
            grid=(_NUM_CORES,),
            in_specs=[
                pl.BlockSpec(memory_space=pl.ANY),
                pl.BlockSpec(w1.shape, lambda c: (0, 0)),
                pl.BlockSpec(w2.shape, lambda c: (0, 0)),
            ],
            out_specs=pl.BlockSpec(memory_space=pl.ANY),
            scratch_shapes=[
                pltpu.VMEM((depth, _CHUNK_B, C, HW), x.dtype),
                pltpu.VMEM((depth, _CHUNK_B, C, HW), x.dtype),
                pltpu.SemaphoreType.DMA((depth,)),
                pltpu.SemaphoreType.DMA((depth,)),
            ],
            compiler_params=pltpu.CompilerParams(
                dimension_semantics=("parallel",),
                vmem_limit_bytes=56 * 1024 * 1024,
            ),
        )(x, w1, w2)
    else:
        bt = 1
        slab = C * HW * x.dtype.itemsize
        while (
            bt * 2 <= 16
            and B % (bt * 2) == 0
            and B // (bt * 2) >= 4
            and 4 * (bt * 2) * slab <= 44 * 1024 * 1024
        ):
            bt *= 2
        out = pl.pallas_call(
            functools.partial(_se_auto, inv_hw=inv_hw),
            out_shape=jax.ShapeDtypeStruct((B, C, HW), x.dtype),
            grid=(B // bt,),
            in_specs=[
                pl.BlockSpec((bt, C, HW), lambda b: (b, 0, 0)),
                pl.BlockSpec(w1.shape, lambda b: (0, 0)),
                pl.BlockSpec(w2.shape, lambda b: (0, 0)),
            ],
            out_specs=pl.BlockSpec((bt, C, HW), lambda b: (b, 0, 0)),
            compiler_params=pltpu.CompilerParams(
                dimension_semantics=("parallel",),
                vmem_limit_bytes=56 * 1024 * 1024,
            ),
        )(x, w1, w2)

    return out.reshape(B, C, H, W)
